# Initial kernel scaffold; baseline (speedup 1.0000x reference)
import jax
import jax.numpy as jnp
from jax.experimental import pallas as pl


def kernel(inputs, pos_table):
    raise NotImplementedError("write your pallas kernel here")


# speedup vs baseline: 12.0884x; 12.0884x over previous
import jax
import jax.numpy as jnp
from jax.experimental import pallas as pl


def kernel(inputs, pos_table):
    N, T = inputs.shape
    H = pos_table.shape[1]
    scale = float(H) ** 0.5
    flat = pos_table.reshape(1, T * H)

    BN = 128

    def body(tab_ref, out_ref):
        out_ref[...] = jnp.broadcast_to(tab_ref[...] * scale, out_ref.shape)

    out = pl.pallas_call(
        body,
        grid=(N // BN,),
        in_specs=[pl.BlockSpec((1, T * H), lambda i: (0, 0))],
        out_specs=pl.BlockSpec((BN, T * H), lambda i: (i, 0)),
        out_shape=jax.ShapeDtypeStruct((N, T * H), jnp.float32),
    )(flat)
    return out.reshape(N, T, H)
